# baseline (speedup 1.0000x reference)
import jax
import jax.numpy as jnp
from jax.experimental import pallas as pl
from jax.experimental.pallas import tpu as pltpu

BT = 512
BF = 1024


def _ffn_body(x_ref, w1_ref, w3_ref, w2_ref, o_ref, xb_ref):
    f = pl.program_id(1)

    @pl.when(f == 0)
    def _pack_x():
        xb_ref[...] = x_ref[...].astype(jnp.bfloat16)

    xb = xb_ref[...]
    a = jnp.dot(xb, w1_ref[...], preferred_element_type=jnp.float32)
    b = jnp.dot(xb, w3_ref[...], preferred_element_type=jnp.float32)
    h = a * b
    act = (h * jax.nn.sigmoid(h)).astype(jnp.bfloat16)
    partial = jnp.dot(act, w2_ref[...], preferred_element_type=jnp.float32)
    o_ref[...] = jnp.where(f == 0, partial, o_ref[...] + partial)


def kernel(hidden_states, W_gate, W1, W3, W2):
    T, D = hidden_states.shape
    F = W1.shape[1]
    nt, nf = T // BT, F // BF
    return pl.pallas_call(
        _ffn_body,
        grid=(nt, nf),
        in_specs=[
            pl.BlockSpec((BT, D), lambda t, f: (t, 0)),
            pl.BlockSpec((D, BF), lambda t, f: (0, f)),
            pl.BlockSpec((D, BF), lambda t, f: (0, f)),
            pl.BlockSpec((BF, D), lambda t, f: (f, 0)),
        ],
        out_specs=pl.BlockSpec((BT, D), lambda t, f: (t, 0)),
        out_shape=jax.ShapeDtypeStruct((T, D), jnp.float32),
        scratch_shapes=[pltpu.VMEM((BT, D), jnp.bfloat16)],
        compiler_params=pltpu.CompilerParams(
            dimension_semantics=("arbitrary", "arbitrary"),
        ),
    )(
        hidden_states,
        W1.astype(jnp.bfloat16),
        W3.astype(jnp.bfloat16),
        W2.astype(jnp.bfloat16),
    )
